# Initial kernel scaffold; baseline (speedup 1.0000x reference)
import jax
import jax.numpy as jnp
from jax.experimental import pallas as pl


def kernel(x_artist, x_cont, x_cat, artist_table, cat_tables, W1, b1, W2, b2, Wp, bp, gamma, beta):
    raise NotImplementedError("write your pallas kernel here")


# speedup vs baseline: 20.3299x; 20.3299x over previous
import functools

import jax
import jax.numpy as jnp
from jax import lax
from jax.experimental import pallas as pl
from jax.experimental.pallas import tpu as pltpu
from jax.experimental.pallas import tpu_sc as plsc

B, L = 4096, 200
NT = B * L
V_ART, D_ART = 1000000, 32
N_CAT, V_CAT, D_PER_CAT = 8, 10000, 4
N_CONT, D_CONT = 9, 64
D_MODEL = 128

NC, NS = 2, 16
NW = NC * NS
TPW = NT // NW
CHUNK = 512
NCHUNK = TPW // CHUNK
ART_ROWS = CHUNK // 128
CAT_ROWS = CHUNK * N_CAT // 128
CAT_W = 8


def _sc_gather(xart2d, xcat2d, artist_table, cat_comb):
  mesh = plsc.VectorSubcoreMesh(core_axis_name="c", subcore_axis_name="s")

  @functools.partial(
      pl.kernel,
      out_type=(
          jax.ShapeDtypeStruct((NT, D_ART), jnp.float32),
          jax.ShapeDtypeStruct((NT * N_CAT, CAT_W), jnp.float32),
      ),
      mesh=mesh,
      scratch_types=[
          pltpu.VMEM((ART_ROWS, 128), jnp.int32),
          pltpu.VMEM((CAT_ROWS, 128), jnp.int32),
          pltpu.VMEM((CAT_ROWS, 128), jnp.int32),
          pltpu.VMEM((CHUNK, D_ART), jnp.float32),
          pltpu.VMEM((CHUNK * N_CAT, CAT_W), jnp.float32),
          pltpu.VMEM_SHARED((N_CAT * V_CAT, CAT_W), jnp.float32),
          pltpu.SemaphoreType.DMA,
          pltpu.SemaphoreType.DMA,
          pltpu.SemaphoreType.DMA,
      ],
      compiler_params=pltpu.CompilerParams(use_tc_tiling_on_sc=False),
  )
  def k(xart_hbm, xcat_hbm, art_hbm, cat_hbm, g_art, g_cat,
        idx_art_v, idx_cat_v, flat_v, art_rows_v, cat_rows_v,
        cat_sp, sem_stage, sem_art, sem_cat):
    cid = lax.axis_index("c")
    sid = lax.axis_index("s")
    wid = sid * NC + cid

    @pl.when(sid == 0)
    def _():
      pltpu.sync_copy(cat_hbm, cat_sp)

    plsc.subcore_barrier()

    off = (lax.iota(jnp.int32, 16) % N_CAT) * V_CAT

    @pl.loop(0, NCHUNK)
    def _chunk(t):
      tok0 = wid * TPW + t * CHUNK
      art_r0 = wid * (TPW // 128) + t * ART_ROWS
      cat_r0 = wid * (TPW * N_CAT // 128) + t * CAT_ROWS

      c1 = pltpu.async_copy(
          xart_hbm.at[pl.ds(art_r0, ART_ROWS)], idx_art_v, sem_stage)
      c2 = pltpu.async_copy(
          xcat_hbm.at[pl.ds(cat_r0, CAT_ROWS)], idx_cat_v, sem_stage)
      c1.wait()
      c2.wait()

      art_cps = [
          pltpu.async_copy(
              art_hbm.at[idx_art_v.at[j]],
              art_rows_v.at[pl.ds(j * 128, 128)],
              sem_art,
          )
          for j in range(ART_ROWS)
      ]

      @pl.loop(0, CAT_ROWS)
      def _flat(j):
        for kk in range(8):
          flat_v[j, pl.ds(kk * 16, 16)] = (
              idx_cat_v[j, pl.ds(kk * 16, 16)] + off)

      @pl.loop(0, CAT_ROWS // 8)
      def _catg(s):
        cps = []
        for kk in range(8):
          j = s * 8 + kk
          cps.append(
              pltpu.async_copy(
                  cat_sp.at[flat_v.at[j]],
                  cat_rows_v.at[pl.ds(j * 128, 128)],
                  sem_cat,
              ))
        for cp in cps:
          cp.wait()

      for cp in art_cps:
        cp.wait()

      pltpu.sync_copy(art_rows_v, g_art.at[pl.ds(tok0, CHUNK)])
      pltpu.sync_copy(
          cat_rows_v, g_cat.at[pl.ds(tok0 * N_CAT, CHUNK * N_CAT)])

  return k(xart2d, xcat2d, artist_table, cat_comb)


def _tc_body(ga_ref, gc_ref, xc_ref, wpa_ref, wpc_ref, w1_ref, b1_ref,
             w2p_ref, bt_ref, gamma_ref, beta_ref, out_ref):
  h = jnp.maximum(
      jnp.dot(xc_ref[...], w1_ref[...], preferred_element_type=jnp.float32)
      + b1_ref[...], 0.0)
  y = (jnp.dot(ga_ref[...], wpa_ref[...], preferred_element_type=jnp.float32)
       + jnp.dot(gc_ref[...], wpc_ref[...], preferred_element_type=jnp.float32)
       + jnp.dot(h, w2p_ref[...], preferred_element_type=jnp.float32)
       + bt_ref[...])
  mu = jnp.mean(y, axis=-1, keepdims=True)
  d = y - mu
  var = jnp.mean(d * d, axis=-1, keepdims=True)
  out_ref[...] = d * lax.rsqrt(var + 1e-5) * gamma_ref[...] + beta_ref[...]


def _tc_project(g_art, g_cat, x_cont2d, wpa, wpc, w1, b1, w2p, btot,
                gamma, beta, t_blk):
  grid = (NT // t_blk,)
  tok_spec = lambda d: pl.BlockSpec((t_blk, d), lambda i: (i, 0))
  full = lambda s: pl.BlockSpec(s, lambda i: tuple(0 for _ in s))
  return pl.pallas_call(
      _tc_body,
      grid=grid,
      in_specs=[
          tok_spec(D_ART),
          tok_spec(N_CAT * CAT_W),
          tok_spec(N_CONT),
          full((D_ART, D_MODEL)),
          full((N_CAT * CAT_W, D_MODEL)),
          full((N_CONT, D_CONT)),
          full((D_CONT,)),
          full((D_CONT, D_MODEL)),
          full((D_MODEL,)),
          full((D_MODEL,)),
          full((D_MODEL,)),
      ],
      out_specs=tok_spec(D_MODEL),
      out_shape=jax.ShapeDtypeStruct((NT, D_MODEL), jnp.float32),
  )(g_art, g_cat, x_cont2d, wpa, wpc, w1, b1, w2p, btot, gamma, beta)


@jax.jit
def kernel(x_artist, x_cont, x_cat, artist_table, cat_tables,
           W1, b1, W2, b2, Wp, bp, gamma, beta):
  xart2d = x_artist.reshape(NT // 128, 128).astype(jnp.int32)
  xcat2d = x_cat.reshape(NT * N_CAT // 128, 128).astype(jnp.int32)
  cat_comb = jnp.pad(
      cat_tables.reshape(N_CAT * V_CAT, D_PER_CAT),
      ((0, 0), (0, CAT_W - D_PER_CAT)))

  g_art, g_cat = _sc_gather(xart2d, xcat2d, artist_table, cat_comb)
  g_cat = g_cat.reshape(NT, N_CAT * CAT_W)

  wpa = Wp[:D_ART]
  wpm = Wp[D_ART:D_ART + D_CONT]
  wpc = Wp[D_ART + D_CONT:]
  wpc = jnp.pad(
      wpc.reshape(N_CAT, D_PER_CAT, D_MODEL),
      ((0, 0), (0, CAT_W - D_PER_CAT), (0, 0))).reshape(
          N_CAT * CAT_W, D_MODEL)
  w2p = W2 @ wpm
  btot = bp + b2 @ wpm

  out = _tc_project(
      g_art, g_cat, x_cont.reshape(NT, N_CONT), wpa, wpc, W1, b1, w2p,
      btot, gamma, beta, t_blk=2048)
  return out.reshape(B, L, D_MODEL)


# speedup vs baseline: 27.8508x; 1.3699x over previous
import functools

import jax
import jax.numpy as jnp
from jax import lax
from jax.experimental import pallas as pl
from jax.experimental.pallas import tpu as pltpu
from jax.experimental.pallas import tpu_sc as plsc

B, L = 4096, 200
NT = B * L
V_ART, D_ART = 1000000, 32
N_CAT, V_CAT, D_PER_CAT = 8, 10000, 4
N_CONT, D_CONT = 9, 64
D_MODEL = 128

NC, NS = 2, 16
NW = NC * NS
BPW = B // NW
LC = 4
NCHUNK = L // LC
CHUNK = LC * BPW
CAT_W = 8
G_CAT0 = D_ART
G_USED = D_ART + N_CAT * CAT_W


def _sc_gather(xart_t, xcat_t, artist_table, cat_comb):
  mesh = plsc.VectorSubcoreMesh(core_axis_name="c", subcore_axis_name="s")

  @functools.partial(
      pl.kernel,
      out_type=jax.ShapeDtypeStruct((B, L, 128), jnp.float32),
      mesh=mesh,
      scratch_types=[
          pltpu.VMEM((LC, 128), jnp.int32),
          pltpu.VMEM((LC * N_CAT, 128), jnp.int32),
          pltpu.VMEM((LC * N_CAT, 128), jnp.int32),
          pltpu.VMEM((CHUNK, D_ART), jnp.float32),
          pltpu.VMEM((CHUNK * N_CAT, CAT_W), jnp.float32),
          pltpu.VMEM_SHARED((N_CAT * V_CAT, CAT_W), jnp.float32),
          pltpu.SemaphoreType.DMA,
          pltpu.SemaphoreType.DMA,
          pltpu.SemaphoreType.DMA,
          pltpu.SemaphoreType.DMA,
      ],
      compiler_params=pltpu.CompilerParams(use_tc_tiling_on_sc=False),
  )
  def k(xart_hbm, xcat_hbm, art_hbm, cat_hbm, g,
        idx_art_v, idx_cat_v, flat_v, art_rows_v, cat_rows_v,
        cat_sp, sem_stage, sem_art, sem_cat, sem_out):
    cid = lax.axis_index("c")
    sid = lax.axis_index("s")
    wid = sid * NC + cid
    b0 = wid * BPW

    @pl.when(sid == 0)
    def _():
      pltpu.sync_copy(cat_hbm, cat_sp)

    plsc.subcore_barrier()

    @pl.loop(0, NCHUNK)
    def _chunk(t):
      l0 = t * LC

      c1 = pltpu.async_copy(
          xart_hbm.at[pl.ds(l0, LC), pl.ds(b0, BPW)], idx_art_v, sem_stage)
      c2 = pltpu.async_copy(
          xcat_hbm.at[pl.ds(l0 * N_CAT, LC * N_CAT), pl.ds(b0, BPW)],
          idx_cat_v, sem_stage)
      c1.wait()
      c2.wait()

      art_cps = [
          pltpu.async_copy(
              art_hbm.at[idx_art_v.at[j]],
              art_rows_v.at[pl.ds(j * BPW, BPW)],
              sem_art,
          )
          for j in range(LC)
      ]

      @pl.loop(0, LC * N_CAT)
      def _flat(j):
        voff = (j % N_CAT) * V_CAT
        for kk in range(8):
          flat_v[j, pl.ds(kk * 16, 16)] = (
              idx_cat_v[j, pl.ds(kk * 16, 16)] + voff)

      @pl.loop(0, LC * N_CAT // 8)
      def _catg(s):
        cps = []
        for kk in range(8):
          j = s * 8 + kk
          cps.append(
              pltpu.async_copy(
                  cat_sp.at[flat_v.at[j]],
                  cat_rows_v.at[pl.ds(j * BPW, BPW)],
                  sem_cat,
              ))
        for cp in cps:
          cp.wait()

      for cp in art_cps:
        cp.wait()

      out_cps = []
      for l_loc in range(LC):
        l_abs = l0 + l_loc
        out_cps.append(pltpu.async_copy(
            art_rows_v.at[pl.ds(l_loc * BPW, BPW)],
            g.at[pl.ds(b0, BPW), l_abs, pl.ds(0, D_ART)],
            sem_out))
        for c in range(N_CAT):
          out_cps.append(pltpu.async_copy(
              cat_rows_v.at[pl.ds((l_loc * N_CAT + c) * BPW, BPW)],
              g.at[pl.ds(b0, BPW), l_abs, pl.ds(G_CAT0 + c * CAT_W, CAT_W)],
              sem_out))
      for cp in out_cps:
        cp.wait()

  return k(xart_t, xcat_t, artist_table, cat_comb)


TBB = 256
TL = 8
TB = TBB * TL


def _tc_body(g_ref, xc_ref, wbig_ref, w1_ref, b1_ref,
             w2p_ref, bt_ref, gamma_ref, beta_ref, out_ref):
  gm = jnp.where(
      lax.broadcasted_iota(jnp.int32, (TB, 128), 1) < G_USED,
      g_ref[...].reshape(TB, 128), 0.0)
  xc = xc_ref[...].reshape(TB, N_CONT)
  h = jnp.maximum(
      jnp.dot(xc, w1_ref[...], preferred_element_type=jnp.float32)
      + b1_ref[...], 0.0)
  y = (jnp.dot(gm, wbig_ref[...], preferred_element_type=jnp.float32)
       + jnp.dot(h, w2p_ref[...], preferred_element_type=jnp.float32)
       + bt_ref[...])
  mu = jnp.mean(y, axis=-1, keepdims=True)
  d = y - mu
  var = jnp.mean(d * d, axis=-1, keepdims=True)
  out_ref[...] = (d * lax.rsqrt(var + 1e-5) * gamma_ref[...]
                  + beta_ref[...]).reshape(TBB, TL, D_MODEL)


def _tc_project(g, xc3, wbig, w1, b1, w2p, btot, gamma, beta):
  grid = (B // TBB, L // TL)
  full = lambda s: pl.BlockSpec(s, lambda i, j: tuple(0 for _ in s))
  return pl.pallas_call(
      _tc_body,
      grid=grid,
      in_specs=[
          pl.BlockSpec((TBB, TL, 128), lambda i, j: (i, j, 0)),
          pl.BlockSpec((TBB, TL, N_CONT), lambda i, j: (i, j, 0)),
          full((128, D_MODEL)),
          full((N_CONT, D_CONT)),
          full((D_CONT,)),
          full((D_CONT, D_MODEL)),
          full((D_MODEL,)),
          full((D_MODEL,)),
          full((D_MODEL,)),
      ],
      out_specs=pl.BlockSpec((TBB, TL, D_MODEL), lambda i, j: (i, j, 0)),
      out_shape=jax.ShapeDtypeStruct((B, L, D_MODEL), jnp.float32),
  )(g, xc3, wbig, w1, b1, w2p, btot, gamma, beta)


@jax.jit
def kernel(x_artist, x_cont, x_cat, artist_table, cat_tables,
           W1, b1, W2, b2, Wp, bp, gamma, beta):
  xart_t = x_artist.T.astype(jnp.int32)
  xcat_t = jnp.transpose(x_cat, (1, 2, 0)).reshape(
      L * N_CAT, B).astype(jnp.int32)
  cat_comb = jnp.pad(
      cat_tables, ((0, 0), (0, 0), (0, CAT_W - D_PER_CAT))).reshape(
          N_CAT * V_CAT, CAT_W)

  g = _sc_gather(xart_t, xcat_t, artist_table, cat_comb)

  xc3 = x_cont

  wpa = Wp[:D_ART]
  wpm = Wp[D_ART:D_ART + D_CONT]
  wpc = Wp[D_ART + D_CONT:]
  wpc_pad = jnp.pad(
      wpc.reshape(N_CAT, D_PER_CAT, D_MODEL),
      ((0, 0), (0, CAT_W - D_PER_CAT), (0, 0))).reshape(
          N_CAT * CAT_W, D_MODEL)
  wbig = jnp.concatenate(
      [wpa, wpc_pad, jnp.zeros((128 - G_USED, D_MODEL), jnp.float32)],
      axis=0)
  w2p = W2 @ wpm
  btot = bp + b2 @ wpm

  return _tc_project(g, xc3, wbig, W1, b1, w2p, btot, gamma, beta)


# speedup vs baseline: 28.9259x; 1.0386x over previous
import functools

import jax
import jax.numpy as jnp
from jax import lax
from jax.experimental import pallas as pl
from jax.experimental.pallas import tpu as pltpu
from jax.experimental.pallas import tpu_sc as plsc

B, L = 4096, 200
NT = B * L
V_ART, D_ART = 1000000, 32
N_CAT, V_CAT, D_PER_CAT = 8, 10000, 4
N_CONT, D_CONT = 9, 64
D_MODEL = 128

NC, NS = 2, 16
NW = NC * NS
BPW = B // NW
LC = 2
NCHUNK = L // LC
CHUNK = LC * BPW
CAT_W = 8
G_CAT0 = D_ART
G_USED = D_ART + N_CAT * CAT_W


def _sc_gather(xart_t, xcat_t, artist_table, cat_comb):
  mesh = plsc.VectorSubcoreMesh(core_axis_name="c", subcore_axis_name="s")

  @functools.partial(
      pl.kernel,
      out_type=jax.ShapeDtypeStruct((B, L, 128), jnp.float32),
      mesh=mesh,
      scratch_types=[
          pltpu.VMEM((LC, 128), jnp.int32),
          pltpu.VMEM((LC * N_CAT, 128), jnp.int32),
          pltpu.VMEM((LC * N_CAT, 128), jnp.int32),
          pltpu.VMEM((CHUNK, D_ART), jnp.float32),
          pltpu.VMEM((CHUNK, D_ART), jnp.float32),
          pltpu.VMEM((CHUNK * N_CAT, CAT_W), jnp.float32),
          pltpu.VMEM((CHUNK * N_CAT, CAT_W), jnp.float32),
          pltpu.VMEM_SHARED((N_CAT * V_CAT, CAT_W), jnp.float32),
          pltpu.SemaphoreType.DMA,
          pltpu.SemaphoreType.DMA,
          pltpu.SemaphoreType.DMA,
          pltpu.SemaphoreType.DMA,
          pltpu.SemaphoreType.DMA,
      ],
      compiler_params=pltpu.CompilerParams(use_tc_tiling_on_sc=False),
  )
  def k(xart_hbm, xcat_hbm, art_hbm, cat_hbm, g,
        idx_art_v, idx_cat_v, flat_v, art_rows_a, art_rows_b,
        cat_rows_a, cat_rows_b, cat_sp,
        sem_stage, sem_art, sem_cat, sem_out_a, sem_out_b):
    cid = lax.axis_index("c")
    sid = lax.axis_index("s")
    wid = sid * NC + cid
    b0 = wid * BPW

    @pl.when(sid == 0)
    def _():
      pltpu.sync_copy(cat_hbm, cat_sp)

    plsc.subcore_barrier()

    def scatter_list(art_rows_v, cat_rows_v, l0, sem_out):
      pairs = []
      for l_loc in range(LC):
        l_abs = l0 + l_loc
        pairs.append((art_rows_v.at[pl.ds(l_loc * BPW, BPW)],
                      g.at[pl.ds(b0, BPW), l_abs, pl.ds(0, D_ART)]))
        for c in range(N_CAT):
          pairs.append((
              cat_rows_v.at[pl.ds((l_loc * N_CAT + c) * BPW, BPW)],
              g.at[pl.ds(b0, BPW), l_abs,
                   pl.ds(G_CAT0 + c * CAT_W, CAT_W)]))
      return [(s, d, sem_out) for (s, d) in pairs]

    def run_chunk(t, art_rows_v, cat_rows_v, sem_out, first):
      l0 = t * LC

      c1 = pltpu.async_copy(
          xart_hbm.at[pl.ds(l0, LC), pl.ds(b0, BPW)], idx_art_v, sem_stage)
      c2 = pltpu.async_copy(
          xcat_hbm.at[pl.ds(l0 * N_CAT, LC * N_CAT), pl.ds(b0, BPW)],
          idx_cat_v, sem_stage)
      c1.wait()
      c2.wait()

      @pl.when(jnp.logical_not(first))
      def _():
        for s, d, sem in scatter_list(art_rows_v, cat_rows_v, l0, sem_out):
          pltpu.make_async_copy(s, d, sem).wait()

      art_cps = [
          pltpu.async_copy(
              art_hbm.at[idx_art_v.at[j]],
              art_rows_v.at[pl.ds(j * BPW, BPW)],
              sem_art,
          )
          for j in range(LC)
      ]

      @pl.loop(0, LC * N_CAT)
      def _flat(j):
        voff = (j % N_CAT) * V_CAT
        for kk in range(8):
          flat_v[j, pl.ds(kk * 16, 16)] = (
              idx_cat_v[j, pl.ds(kk * 16, 16)] + voff)

      cat_cps = [
          pltpu.async_copy(
              cat_sp.at[flat_v.at[j]],
              cat_rows_v.at[pl.ds(j * BPW, BPW)],
              sem_cat,
          )
          for j in range(LC * N_CAT)
      ]
      for cp in cat_cps:
        cp.wait()
      for cp in art_cps:
        cp.wait()

      for s, d, sem in scatter_list(art_rows_v, cat_rows_v, l0, sem_out):
        pltpu.async_copy(s, d, sem)

    @pl.loop(0, NCHUNK // 2)
    def _chunk2(i):
      run_chunk(i * 2, art_rows_a, cat_rows_a, sem_out_a, i == 0)
      run_chunk(i * 2 + 1, art_rows_b, cat_rows_b, sem_out_b, i == 0)

    for art_rows_v, cat_rows_v, sem_out in (
        (art_rows_a, cat_rows_a, sem_out_a),
        (art_rows_b, cat_rows_b, sem_out_b)):
      for s, d, sem in scatter_list(art_rows_v, cat_rows_v, 0, sem_out):
        pltpu.make_async_copy(s, d, sem).wait()

  return k(xart_t, xcat_t, artist_table, cat_comb)


TBB = 256
TL = 8
TB = TBB * TL


def _tc_body(g_ref, xc_ref, wbig_ref, w1_ref, b1_ref,
             w2p_ref, bt_ref, gamma_ref, beta_ref, out_ref):
  gm = jnp.where(
      lax.broadcasted_iota(jnp.int32, (TB, 128), 1) < G_USED,
      g_ref[...].reshape(TB, 128), 0.0)
  xc = xc_ref[...].reshape(TB, N_CONT)
  h = jnp.maximum(
      jnp.dot(xc, w1_ref[...], preferred_element_type=jnp.float32)
      + b1_ref[...], 0.0)
  y = (jnp.dot(gm, wbig_ref[...], preferred_element_type=jnp.float32)
       + jnp.dot(h, w2p_ref[...], preferred_element_type=jnp.float32)
       + bt_ref[...])
  mu = jnp.mean(y, axis=-1, keepdims=True)
  d = y - mu
  var = jnp.mean(d * d, axis=-1, keepdims=True)
  out_ref[...] = (d * lax.rsqrt(var + 1e-5) * gamma_ref[...]
                  + beta_ref[...]).reshape(TBB, TL, D_MODEL)


def _tc_project(g, xc3, wbig, w1, b1, w2p, btot, gamma, beta):
  grid = (B // TBB, L // TL)
  full = lambda s: pl.BlockSpec(s, lambda i, j: tuple(0 for _ in s))
  return pl.pallas_call(
      _tc_body,
      grid=grid,
      in_specs=[
          pl.BlockSpec((TBB, TL, 128), lambda i, j: (i, j, 0)),
          pl.BlockSpec((TBB, TL, N_CONT), lambda i, j: (i, j, 0)),
          full((128, D_MODEL)),
          full((N_CONT, D_CONT)),
          full((D_CONT,)),
          full((D_CONT, D_MODEL)),
          full((D_MODEL,)),
          full((D_MODEL,)),
          full((D_MODEL,)),
      ],
      out_specs=pl.BlockSpec((TBB, TL, D_MODEL), lambda i, j: (i, j, 0)),
      out_shape=jax.ShapeDtypeStruct((B, L, D_MODEL), jnp.float32),
  )(g, xc3, wbig, w1, b1, w2p, btot, gamma, beta)


@jax.jit
def kernel(x_artist, x_cont, x_cat, artist_table, cat_tables,
           W1, b1, W2, b2, Wp, bp, gamma, beta):
  xart_t = x_artist.T.astype(jnp.int32)
  xcat_t = jnp.transpose(x_cat, (1, 2, 0)).reshape(
      L * N_CAT, B).astype(jnp.int32)
  cat_comb = jnp.pad(
      cat_tables, ((0, 0), (0, 0), (0, CAT_W - D_PER_CAT))).reshape(
          N_CAT * V_CAT, CAT_W)

  g = _sc_gather(xart_t, xcat_t, artist_table, cat_comb)

  xc3 = x_cont

  wpa = Wp[:D_ART]
  wpm = Wp[D_ART:D_ART + D_CONT]
  wpc = Wp[D_ART + D_CONT:]
  wpc_pad = jnp.pad(
      wpc.reshape(N_CAT, D_PER_CAT, D_MODEL),
      ((0, 0), (0, CAT_W - D_PER_CAT), (0, 0))).reshape(
          N_CAT * CAT_W, D_MODEL)
  wbig = jnp.concatenate(
      [wpa, wpc_pad, jnp.zeros((128 - G_USED, D_MODEL), jnp.float32)],
      axis=0)
  w2p = W2 @ wpm
  btot = bp + b2 @ wpm

  return _tc_project(g, xc3, wbig, W1, b1, w2p, btot, gamma, beta)


# speedup vs baseline: 29.3901x; 1.0160x over previous
import functools

import jax
import jax.numpy as jnp
from jax import lax
from jax.experimental import pallas as pl
from jax.experimental.pallas import tpu as pltpu
from jax.experimental.pallas import tpu_sc as plsc

B, L = 4096, 200
NT = B * L
V_ART, D_ART = 1000000, 32
N_CAT, V_CAT, D_PER_CAT = 8, 10000, 4
N_CONT, D_CONT = 9, 64
D_MODEL = 128

NC, NS = 2, 16
NW = NC * NS
BPW = B // NW
LC = 2
NCHUNK = L // LC
CHUNK = LC * BPW
CAT_W = 8
G_CAT0 = D_ART
G_USED = D_ART + N_CAT * CAT_W


def _sc_gather(xart_t, xcat_t, artist_table, cat_comb, lh):
  mesh = plsc.VectorSubcoreMesh(core_axis_name="c", subcore_axis_name="s")
  nchunk = lh // LC

  @functools.partial(
      pl.kernel,
      out_type=jax.ShapeDtypeStruct((B, lh, 128), jnp.float32),
      mesh=mesh,
      scratch_types=[
          pltpu.VMEM((LC, 128), jnp.int32),
          pltpu.VMEM((LC * N_CAT, 128), jnp.int32),
          pltpu.VMEM((LC * N_CAT, 128), jnp.int32),
          pltpu.VMEM((CHUNK, D_ART), jnp.float32),
          pltpu.VMEM((CHUNK, D_ART), jnp.float32),
          pltpu.VMEM((CHUNK * N_CAT, CAT_W), jnp.float32),
          pltpu.VMEM((CHUNK * N_CAT, CAT_W), jnp.float32),
          pltpu.VMEM_SHARED((N_CAT * V_CAT, CAT_W), jnp.float32),
          pltpu.SemaphoreType.DMA,
          pltpu.SemaphoreType.DMA,
          pltpu.SemaphoreType.DMA,
          pltpu.SemaphoreType.DMA,
          pltpu.SemaphoreType.DMA,
      ],
      compiler_params=pltpu.CompilerParams(use_tc_tiling_on_sc=False),
  )
  def k(xart_hbm, xcat_hbm, art_hbm, cat_hbm, g,
        idx_art_v, idx_cat_v, flat_v, art_rows_a, art_rows_b,
        cat_rows_a, cat_rows_b, cat_sp,
        sem_stage, sem_art, sem_cat, sem_out_a, sem_out_b):
    cid = lax.axis_index("c")
    sid = lax.axis_index("s")
    wid = sid * NC + cid
    b0 = wid * BPW

    @pl.when(sid == 0)
    def _():
      pltpu.sync_copy(cat_hbm, cat_sp)

    plsc.subcore_barrier()

    def scatter_list(art_rows_v, cat_rows_v, l0, sem_out):
      pairs = []
      for l_loc in range(LC):
        l_abs = l0 + l_loc
        pairs.append((art_rows_v.at[pl.ds(l_loc * BPW, BPW)],
                      g.at[pl.ds(b0, BPW), l_abs, pl.ds(0, D_ART)]))
        for c in range(N_CAT):
          pairs.append((
              cat_rows_v.at[pl.ds((l_loc * N_CAT + c) * BPW, BPW)],
              g.at[pl.ds(b0, BPW), l_abs,
                   pl.ds(G_CAT0 + c * CAT_W, CAT_W)]))
      return [(s, d, sem_out) for (s, d) in pairs]

    def run_chunk(t, art_rows_v, cat_rows_v, sem_out, first):
      l0 = t * LC

      c1 = pltpu.async_copy(
          xart_hbm.at[pl.ds(l0, LC), pl.ds(b0, BPW)], idx_art_v, sem_stage)
      c2 = pltpu.async_copy(
          xcat_hbm.at[pl.ds(l0 * N_CAT, LC * N_CAT), pl.ds(b0, BPW)],
          idx_cat_v, sem_stage)
      c1.wait()
      c2.wait()

      @pl.when(jnp.logical_not(first))
      def _():
        for s, d, sem in scatter_list(art_rows_v, cat_rows_v, l0, sem_out):
          pltpu.make_async_copy(s, d, sem).wait()

      art_cps = [
          pltpu.async_copy(
              art_hbm.at[idx_art_v.at[j]],
              art_rows_v.at[pl.ds(j * BPW, BPW)],
              sem_art,
          )
          for j in range(LC)
      ]

      @pl.loop(0, LC * N_CAT)
      def _flat(j):
        voff = (j % N_CAT) * V_CAT
        for kk in range(8):
          flat_v[j, pl.ds(kk * 16, 16)] = (
              idx_cat_v[j, pl.ds(kk * 16, 16)] + voff)

      cat_cps = [
          pltpu.async_copy(
              cat_sp.at[flat_v.at[j]],
              cat_rows_v.at[pl.ds(j * BPW, BPW)],
              sem_cat,
          )
          for j in range(LC * N_CAT)
      ]
      for cp in cat_cps:
        cp.wait()
      for cp in art_cps:
        cp.wait()

      for s, d, sem in scatter_list(art_rows_v, cat_rows_v, l0, sem_out):
        pltpu.async_copy(s, d, sem)

    @pl.loop(0, nchunk // 2)
    def _chunk2(i):
      run_chunk(i * 2, art_rows_a, cat_rows_a, sem_out_a, i == 0)
      run_chunk(i * 2 + 1, art_rows_b, cat_rows_b, sem_out_b, i == 0)

    for art_rows_v, cat_rows_v, sem_out in (
        (art_rows_a, cat_rows_a, sem_out_a),
        (art_rows_b, cat_rows_b, sem_out_b)):
      for s, d, sem in scatter_list(art_rows_v, cat_rows_v, 0, sem_out):
        pltpu.make_async_copy(s, d, sem).wait()

  return k(xart_t, xcat_t, artist_table, cat_comb)


TBB = 256
TL = 8
TB = TBB * TL
LH = 96


def _tc_body(g_ref, xc_ref, wbig_ref, w1_ref, b1_ref,
             w2p_ref, bt_ref, gamma_ref, beta_ref, out_ref):
  gm = jnp.where(
      lax.broadcasted_iota(jnp.int32, (TB, 128), 1) < G_USED,
      g_ref[...].reshape(TB, 128), 0.0)
  xc = xc_ref[...].reshape(TB, N_CONT)
  h = jnp.maximum(
      jnp.dot(xc, w1_ref[...], preferred_element_type=jnp.float32)
      + b1_ref[...], 0.0)
  y = (jnp.dot(gm, wbig_ref[...], preferred_element_type=jnp.float32)
       + jnp.dot(h, w2p_ref[...], preferred_element_type=jnp.float32)
       + bt_ref[...])
  mu = jnp.mean(y, axis=-1, keepdims=True)
  d = y - mu
  var = jnp.mean(d * d, axis=-1, keepdims=True)
  out_ref[...] = (d * lax.rsqrt(var + 1e-5) * gamma_ref[...]
                  + beta_ref[...]).reshape(TBB, TL, D_MODEL)


def _tc_project_half(g, xc3, wbig, w1, b1, w2p, btot, gamma, beta,
                     l_off, prev):
  grid = (B // TBB, g.shape[1] // TL)
  jt = l_off // TL
  full = lambda s: pl.BlockSpec(s, lambda i, j: tuple(0 for _ in s))
  in_specs = [
      pl.BlockSpec((TBB, TL, 128), lambda i, j: (i, j, 0)),
      pl.BlockSpec((TBB, TL, N_CONT), lambda i, j: (i, j, 0)),
      full((128, D_MODEL)),
      full((N_CONT, D_CONT)),
      full((D_CONT,)),
      full((D_CONT, D_MODEL)),
      full((D_MODEL,)),
      full((D_MODEL,)),
      full((D_MODEL,)),
  ]
  args = [g, xc3, wbig, w1, b1, w2p, btot, gamma, beta]
  aliases = {}
  body = _tc_body
  if prev is not None:
    in_specs.append(pl.BlockSpec(memory_space=pl.ANY))
    args.append(prev)
    aliases = {9: 0}
    body = lambda *refs: _tc_body(*refs[:9], refs[10])
  return pl.pallas_call(
      body,
      grid=grid,
      in_specs=in_specs,
      out_specs=pl.BlockSpec((TBB, TL, D_MODEL), lambda i, j: (i, jt + j, 0)),
      out_shape=jax.ShapeDtypeStruct((B, L, D_MODEL), jnp.float32),
      input_output_aliases=aliases,
  )(*args)


@jax.jit
def kernel(x_artist, x_cont, x_cat, artist_table, cat_tables,
           W1, b1, W2, b2, Wp, bp, gamma, beta):
  xart_t = x_artist.T.astype(jnp.int32)
  xcat_t = jnp.transpose(x_cat, (1, 2, 0)).reshape(
      L * N_CAT, B).astype(jnp.int32)
  cat_comb = jnp.pad(
      cat_tables, ((0, 0), (0, 0), (0, CAT_W - D_PER_CAT))).reshape(
          N_CAT * V_CAT, CAT_W)

  xc3 = x_cont

  wpa = Wp[:D_ART]
  wpm = Wp[D_ART:D_ART + D_CONT]
  wpc = Wp[D_ART + D_CONT:]
  wpc_pad = jnp.pad(
      wpc.reshape(N_CAT, D_PER_CAT, D_MODEL),
      ((0, 0), (0, CAT_W - D_PER_CAT), (0, 0))).reshape(
          N_CAT * CAT_W, D_MODEL)
  wbig = jnp.concatenate(
      [wpa, wpc_pad, jnp.zeros((128 - G_USED, D_MODEL), jnp.float32)],
      axis=0)
  w2p = W2 @ wpm
  btot = bp + b2 @ wpm

  g0 = _sc_gather(xart_t[:LH], xcat_t[:LH * N_CAT], artist_table,
                  cat_comb, LH)
  g1 = _sc_gather(xart_t[LH:], xcat_t[LH * N_CAT:], artist_table,
                  cat_comb, L - LH)
  y0 = _tc_project_half(g0, xc3[:, :LH], wbig, W1, b1, w2p, btot,
                        gamma, beta, 0, None)
  y1 = _tc_project_half(g1, xc3[:, LH:], wbig, W1, b1, w2p, btot,
                        gamma, beta, LH, y0)
  return y1
